# baseline (speedup 1.0000x reference)
import numpy as np
import jax
import jax.numpy as jnp
from jax import lax
from jax.experimental import pallas as pl
from jax.experimental.pallas import tpu as pltpu

_MAJOR = np.array([30, 32, 33, 46, 47, 48, 50, 54, 76, 82, 84, 88, 90], dtype=np.int32)
_ACC = np.array([10, 11, 16, 20, 24, 35, 36, 53, 62, 63, 71, 73, 74, 77, 85, 93], dtype=np.int32)
_P = 99
_E = 128
_NPOS = len(_MAJOR) + len(_ACC)
_NSEL = 32
_CB0 = 1
_NCB = 15


def _sel_matrix() -> np.ndarray:
    sel = np.zeros((_P * 22, _NSEL), np.float32)
    for j, pos in enumerate(np.concatenate([_MAJOR, _ACC])):
        q = int(pos) - 1
        sel[22 * q: 22 * q + 21, j] = 1.0
    return sel


_SEL = _sel_matrix()[_CB0 * 128: (_CB0 + _NCB) * 128]


def _body(x_ref, sel_ref, mt_ref, at_ref, w_ref, b_ref, out_ref, acc_ref):
    j = pl.program_id(1)
    xb = x_ref[...].astype(jnp.bfloat16)
    part = lax.dot_general(xb, sel_ref[...], (((1,), (0,)), ((), ())),
                           preferred_element_type=jnp.float32)

    @pl.when(j == 0)
    def _():
        acc_ref[...] = part

    @pl.when(j > 0)
    def _():
        acc_ref[...] += part

    @pl.when(j == _NCB - 1)
    def _():
        mut = (acc_ref[...] > 0).astype(jnp.float32)
        pm = lax.dot_general(mt_ref[...], w_ref[:, :_E], (((1,), (1,)), ((), ())),
                             preferred_element_type=jnp.float32)
        pa = lax.dot_general(at_ref[...], w_ref[:, _E:], (((1,), (1,)), ((), ())),
                             preferred_element_type=jnp.float32)
        proj = jnp.concatenate(
            [pm, pa, jnp.zeros((_NSEL - _NPOS, _E), jnp.float32)], axis=0)
        out_ref[...] = lax.dot_general(mut, proj, (((1,), (0,)), ((), ())),
                                       preferred_element_type=jnp.float32) + b_ref[...]


def kernel(x, major_table, accessory_table, W, b):
    batch, feat = x.shape
    bb = 512
    grid = (batch // bb, _NCB)
    sel = jnp.asarray(_SEL, dtype=jnp.bfloat16)
    b2 = b.reshape(1, _E)
    return pl.pallas_call(
        _body,
        grid=grid,
        in_specs=[
            pl.BlockSpec((bb, 128), lambda i, j: (i, j + _CB0)),
            pl.BlockSpec((128, _NSEL), lambda i, j: (j, 0)),
            pl.BlockSpec(major_table.shape, lambda i, j: (0, 0)),
            pl.BlockSpec(accessory_table.shape, lambda i, j: (0, 0)),
            pl.BlockSpec(W.shape, lambda i, j: (0, 0)),
            pl.BlockSpec((1, _E), lambda i, j: (0, 0)),
        ],
        out_specs=pl.BlockSpec((bb, _E), lambda i, j: (i, 0)),
        out_shape=jax.ShapeDtypeStruct((batch, _E), jnp.float32),
        scratch_shapes=[pltpu.VMEM((bb, _NSEL), jnp.float32)],
        compiler_params=pltpu.CompilerParams(
            dimension_semantics=("parallel", "arbitrary"),
        ),
    )(x, sel, major_table, accessory_table, W, b2)
